# baseline (speedup 1.0000x reference)
import functools

import jax
import jax.numpy as jnp
from jax import lax
from jax.experimental import pallas as pl
from jax.experimental.pallas import tpu as pltpu
from jax.experimental.pallas import tpu_sc as plsc

BATCH = 16384
EMBED = 32
LANES = 16
NC = 2
NS = 16
NW = NC * NS
BPW = BATCH // NW
CHUNK = 128
NCH = BPW // CHUNK


@functools.cache
def _build_kernel():
    return functools.partial(
        pl.kernel,
        out_type=jax.ShapeDtypeStruct((BATCH,), jnp.float32),
        mesh=plsc.VectorSubcoreMesh(core_axis_name="c", subcore_axis_name="s"),
        compiler_params=pltpu.CompilerParams(
            needs_layout_passes=False, use_tc_tiling_on_sc=False),
        scratch_types=[
            pltpu.VMEM((NCH, CHUNK), jnp.int32),
            pltpu.VMEM((NCH, CHUNK), jnp.int32),
            pltpu.VMEM((NCH, CHUNK, EMBED), jnp.float32),
            pltpu.VMEM((NCH, CHUNK, EMBED), jnp.float32),
            pltpu.VMEM((LANES * BPW,), jnp.float32),
            pltpu.VMEM((BPW,), jnp.float32),
            pltpu.SemaphoreType.DMA,
        ],
    )(_mf_body)


def _mf_body(qids, mids, qtab, mtab, out, qidx, midx, qrows, mrows, tr,
             outv, sem):
    wid = lax.axis_index("c") * NS + lax.axis_index("s")
    base = wid * BPW

    for j in range(NCH):
        pltpu.sync_copy(qids.at[pl.ds(base + j * CHUNK, CHUNK)], qidx.at[j])
        pltpu.sync_copy(mids.at[pl.ds(base + j * CHUNK, CHUNK)], midx.at[j])

    handles = []
    for j in range(NCH):
        handles.append(pltpu.async_copy(qtab.at[qidx.at[j]], qrows.at[j], sem))
        handles.append(pltpu.async_copy(mtab.at[midx.at[j]], mrows.at[j], sem))
    for h in handles:
        h.wait()

    lane_off = lax.iota(jnp.int32, LANES) * BPW

    for j in range(NCH):
        qr = qrows.at[j]
        mr = mrows.at[j]

        def body(r, carry, j=j, qr=qr, mr=mr):
            b = j * CHUNK + r
            q0 = qr[r, 0:16]
            q1 = qr[r, 16:32]
            m0 = mr[r, 0:16]
            m1 = mr[r, 16:32]
            s = q0 * m0 + q1 * m1
            plsc.store_scatter(tr, [lane_off + b], s)
            return carry

        lax.fori_loop(0, CHUNK, body, 0)

    for g in range(BPW // LANES):
        acc = tr[pl.ds(g * LANES, LANES)]
        for d in range(1, LANES):
            acc = acc + tr[pl.ds(d * BPW + g * LANES, LANES)]
        outv[pl.ds(g * LANES, LANES)] = acc

    pltpu.sync_copy(outv, out.at[pl.ds(base, BPW)])


@jax.jit
def kernel(query_ids, model_ids, query_table, model_table):
    return _build_kernel()(query_ids.astype(jnp.int32),
                           model_ids.astype(jnp.int32),
                           query_table, model_table)
